# baseline (speedup 1.0000x reference)
import functools

import jax
import jax.numpy as jnp
from jax import lax
from jax.experimental import pallas as pl
from jax.experimental.pallas import tpu as pltpu
from jax.experimental.pallas import tpu_sc as plsc

B = 4096
L = 200
D = 64
HID = 128
NUM_CLASSES = 2

NC = 2
NS = 16
NW = NC * NS
B_PER_W = B // NW
GROUPS = D // 16
ROW_UNROLL = 2

_mesh = plsc.VectorSubcoreMesh(core_axis_name="c", subcore_axis_name="s")

VOCAB_N = 1000000
FULL_BLOCKS = VOCAB_N // 128
REM = VOCAB_N - FULL_BLOCKS * 128
_IOTA16 = None


def _splat(v):
    return jnp.broadcast_to(v, (16,)).astype(jnp.int32)


@functools.partial(
    pl.kernel,
    out_type=jax.ShapeDtypeStruct((VOCAB_N, 2 * D), jnp.float32),
    mesh=_mesh,
    compiler_params=pltpu.CompilerParams(needs_layout_passes=False),
    scratch_types=[
        pltpu.VMEM((D, 128), jnp.float32),
        pltpu.VMEM((D, 128), jnp.float32),
        pltpu.VMEM((D, 128), jnp.float32),
        pltpu.VMEM((D, 128), jnp.float32),
        pltpu.VMEM((128, 2 * D), jnp.float32),
        pltpu.VMEM((128, 2 * D), jnp.float32),
        pltpu.VMEM((128, 2 * D), jnp.float32),
        pltpu.VMEM((128, 2 * D), jnp.float32),
        pltpu.VMEM((D, REM), jnp.float32),
        pltpu.VMEM((D, 128), jnp.float32),
        pltpu.SemaphoreType.DMA,
        pltpu.SemaphoreType.DMA,
        pltpu.SemaphoreType.DMA,
        pltpu.SemaphoreType.DMA,
        pltpu.SemaphoreType.DMA,
        pltpu.SemaphoreType.DMA,
        pltpu.SemaphoreType.DMA,
        pltpu.SemaphoreType.DMA,
    ],
)
def _relayout(embt_hbm, tailt_hbm, out_hbm, vin0, vin1, vin2, vin3,
              vout0, vout1, vout2, vout3, vtail, vskew,
              si0, si1, si2, si3, so0, so1, so2, so3):
    wid = lax.axis_index("s") * NC + lax.axis_index("c")
    n_blk = FULL_BLOCKS // NW + jnp.where(wid < FULL_BLOCKS % NW, 1, 0)
    first = wid * (FULL_BLOCKS // NW) + jnp.minimum(wid, FULL_BLOCKS % NW)

    vins = (vin0, vin1, vin2, vin3)
    vouts = (vout0, vout1, vout2, vout3)
    sis = (si0, si1, si2, si3)
    sos = (so0, so1, so2, so3)
    nbuf = 4
    iota = lax.iota(jnp.int32, 16)

    def in_copy(i, p):
        off = pl.multiple_of((first + i) * 128, 128)
        return pltpu.make_async_copy(
            embt_hbm.at[:, pl.ds(off, 128)], vins[p], sis[p]
        )

    def out_copy(i, p):
        off = pl.multiple_of((first + i) * 128, 128)
        return pltpu.make_async_copy(
            vouts[p], out_hbm.at[pl.ds(off, 128)], sos[p]
        )

    def transpose(vin, vout, ncols):
        rvs = [iota + g * 16 for g in range(GROUPS)]

        def skew_body(dt, _):
            for du in range(4):
                d = dt * 4 + du
                rot = (iota + d) & 15
                vals = [
                    vin[d, pl.ds(cg * 16, 16)].at[rot].get(
                        mode="promise_in_bounds")
                    for cg in range(ncols // 16)
                ]
                for cg in range(ncols // 16):
                    vskew[d, pl.ds(cg * 16, 16)] = vals[cg]
            return 0

        lax.fori_loop(0, D // 4, skew_body, 0)

        def j_body(jt, _):
            for ju in range(2):
                j = jt * 2 + ju
                cvec = ((j - iota) & 15) + (j & 0x70)
                vals = [
                    plsc.load_gather(vskew, [rvs[g], cvec])
                    for g in range(GROUPS)
                ]
                for g in range(GROUPS):
                    vout[j, pl.ds(g * 16, 16)] = vals[g]
            return 0

        lax.fori_loop(0, ncols // 2, j_body, 0)

    for p in range(nbuf):
        in_copy(p, p).start()

    def t_body(t, _):
        for p in range(nbuf):
            i = t * nbuf + p

            @pl.when(i < n_blk)
            def _():
                in_copy(i, p).wait()

                @pl.when(i >= nbuf)
                def _():
                    out_copy(i - nbuf, p).wait()

                transpose(vins[p], vouts[p], 128)
                out_copy(i, p).start()

                @pl.when(i + nbuf < n_blk)
                def _():
                    in_copy(i + nbuf, p).start()

        return 0

    max_groups = (FULL_BLOCKS // NW + 1 + nbuf - 1) // nbuf
    lax.fori_loop(0, max_groups, t_body, 0)
    for p in range(nbuf):
        out_copy(p, p).wait()

    @pl.when(wid == NW - 1)
    def _():
        pltpu.sync_copy(tailt_hbm, vtail)
        transpose(vtail, vout0, REM)
        pltpu.sync_copy(
            vout0.at[pl.ds(0, REM)],
            out_hbm.at[pl.ds(FULL_BLOCKS * 128, REM)],
        )


@functools.partial(
    pl.kernel,
    out_type=jax.ShapeDtypeStruct((B, D), jnp.float32),
    mesh=_mesh,
    scratch_types=[
        pltpu.VMEM((L, B_PER_W), jnp.int32),
        pltpu.VMEM((B_PER_W, 2 * D), jnp.float32),
        pltpu.VMEM((B_PER_W, 2 * D), jnp.float32),
        pltpu.VMEM((B_PER_W, D), jnp.float32),
        pltpu.SemaphoreType.DMA,
        pltpu.SemaphoreType.DMA,
    ],
)
def _pool(xt_hbm, embp_hbm, out_hbm, idx_v, buf0, buf1, acc_v, sem0, sem1):
    wid = lax.axis_index("s") * NC + lax.axis_index("c")
    base = wid * B_PER_W

    pltpu.sync_copy(xt_hbm.at[:, pl.ds(base, B_PER_W)], idx_v)

    bufs = (buf0, buf1)
    sems = (sem0, sem1)

    def start(pos, half):
        pltpu.make_async_copy(
            embp_hbm.at[idx_v.at[pos]], bufs[half], sems[half]
        ).start()

    def wait(pos, half):
        pltpu.make_async_copy(
            embp_hbm.at[idx_v.at[pos]], bufs[half], sems[half]
        ).wait()

    zero = jnp.zeros((16,), jnp.float32)

    def zero_body(r, _):
        for u in range(ROW_UNROLL):
            row = r * ROW_UNROLL + u
            for g in range(GROUPS):
                acc_v[row, pl.ds(g * 16, 16)] = zero
        return 0

    lax.fori_loop(0, B_PER_W // ROW_UNROLL, zero_body, 0)

    start(0, 0)
    start(1, 1)

    def accum(buf):
        def row_body(r, _):
            for u in range(ROW_UNROLL):
                row = r * ROW_UNROLL + u
                for g in range(GROUPS):
                    plsc.addupdate(
                        acc_v.at[row, pl.ds(g * 16, 16)],
                        buf[row, pl.ds(g * 16, 16)],
                    )
            return 0

        lax.fori_loop(0, B_PER_W // ROW_UNROLL, row_body, 0)

    def pos_body(i, _):
        for half in range(2):
            pos = i * 2 + half
            wait(pos, half)
            accum(bufs[half])

            @pl.when(i < L // 2 - 1)
            def _():
                start(pos + 2, half)

        return 0

    lax.fori_loop(0, L // 2, pos_body, 0)

    pltpu.sync_copy(acc_v, out_hbm.at[pl.ds(base, B_PER_W)])


def _mlp_body(m_ref, vw_ref, vb_ref, ww_ref, wb_ref, out_ref):
    m = m_ref[...]
    h = jnp.dot(m, vw_ref[...], preferred_element_type=jnp.float32)
    h = jnp.maximum(h + vb_ref[...], 0.0)
    logits = jnp.dot(h, ww_ref[...], preferred_element_type=jnp.float32)
    logits = logits + wb_ref[...]
    mx = jnp.max(logits, axis=1, keepdims=True)
    s = logits - mx
    lse = jnp.log(jnp.sum(jnp.exp(s), axis=1, keepdims=True))
    out_ref[...] = s - lse


def _mlp(m, vw_t, vb, ww_t, wb):
    return pl.pallas_call(
        _mlp_body,
        out_shape=jax.ShapeDtypeStruct((B, NUM_CLASSES), jnp.float32),
    )(m, vw_t, vb, ww_t, wb)


@jax.jit
def kernel(x, emb, V_w, V_b, W_w, W_b):
    xt = x.astype(jnp.int32).T
    tailt = emb[VOCAB_N - REM:].T
    embp = _relayout(emb.T, tailt)
    m_sum = _pool(xt, embp)
    vw_t = V_w.T * jnp.float32(1.0 / L)
    return _mlp(m_sum, vw_t, V_b.reshape(1, HID), W_w.T, W_b.reshape(1, NUM_CLASSES))


# speedup vs baseline: 1.1295x; 1.1295x over previous
import functools

import jax
import jax.numpy as jnp
from jax import lax
from jax.experimental import pallas as pl
from jax.experimental.pallas import tpu as pltpu
from jax.experimental.pallas import tpu_sc as plsc

B = 4096
L = 200
D = 64
HID = 128
NUM_CLASSES = 2

NC = 2
NS = 16
NW = NC * NS
B_PER_W = B // NW
GROUPS = D // 16
ROW_UNROLL = 2

_mesh = plsc.VectorSubcoreMesh(core_axis_name="c", subcore_axis_name="s")

VOCAB_N = 1000000
FULL_BLOCKS = VOCAB_N // 128
REM = VOCAB_N - FULL_BLOCKS * 128
_IOTA16 = None


def _splat(v):
    return jnp.broadcast_to(v, (16,)).astype(jnp.int32)


@functools.partial(
    pl.kernel,
    out_type=jax.ShapeDtypeStruct((VOCAB_N, 2 * D), jnp.float32),
    mesh=_mesh,
    compiler_params=pltpu.CompilerParams(needs_layout_passes=False),
    scratch_types=[
        pltpu.VMEM((D, 128), jnp.float32),
        pltpu.VMEM((D, 128), jnp.float32),
        pltpu.VMEM((D, 128), jnp.float32),
        pltpu.VMEM((D, 128), jnp.float32),
        pltpu.VMEM((128, 2 * D), jnp.float32),
        pltpu.VMEM((128, 2 * D), jnp.float32),
        pltpu.VMEM((128, 2 * D), jnp.float32),
        pltpu.VMEM((128, 2 * D), jnp.float32),
        pltpu.VMEM((D, REM), jnp.float32),
        pltpu.VMEM((D, 128), jnp.float32),
        pltpu.SemaphoreType.DMA,
        pltpu.SemaphoreType.DMA,
        pltpu.SemaphoreType.DMA,
        pltpu.SemaphoreType.DMA,
        pltpu.SemaphoreType.DMA,
        pltpu.SemaphoreType.DMA,
        pltpu.SemaphoreType.DMA,
        pltpu.SemaphoreType.DMA,
    ],
)
def _relayout(embt_hbm, tailt_hbm, out_hbm, vin0, vin1, vin2, vin3,
              vout0, vout1, vout2, vout3, vtail, vskew,
              si0, si1, si2, si3, so0, so1, so2, so3):
    wid = lax.axis_index("s") * NC + lax.axis_index("c")
    n_blk = FULL_BLOCKS // NW + jnp.where(wid < FULL_BLOCKS % NW, 1, 0)
    first = wid * (FULL_BLOCKS // NW) + jnp.minimum(wid, FULL_BLOCKS % NW)

    vins = (vin0, vin1, vin2, vin3)
    vouts = (vout0, vout1, vout2, vout3)
    sis = (si0, si1, si2, si3)
    sos = (so0, so1, so2, so3)
    nbuf = 4
    iota = lax.iota(jnp.int32, 16)

    def in_copy(i, p):
        off = pl.multiple_of((first + i) * 128, 128)
        return pltpu.make_async_copy(
            embt_hbm.at[:, pl.ds(off, 128)], vins[p], sis[p]
        )

    def out_copy(i, p):
        off = pl.multiple_of((first + i) * 128, 128)
        return pltpu.make_async_copy(
            vouts[p], out_hbm.at[pl.ds(off, 128)], sos[p]
        )

    def transpose(vin, vout, ncols):
        rvs = [iota + g * 16 for g in range(GROUPS)]

        def skew_body(dt, _):
            for du in range(4):
                d = dt * 4 + du
                rot = (iota + d) & 15
                vals = [
                    vin[d, pl.ds(cg * 16, 16)].at[rot].get(
                        mode="promise_in_bounds")
                    for cg in range(ncols // 16)
                ]
                for cg in range(ncols // 16):
                    vskew[d, pl.ds(cg * 16, 16)] = vals[cg]
            return 0

        lax.fori_loop(0, D // 4, skew_body, 0)

        def j_body(jt, _):
            for ju in range(2):
                j = jt * 2 + ju
                cvec = ((j - iota) & 15) + (j & 0x70)
                vals = [
                    plsc.load_gather(vskew, [rvs[g], cvec])
                    for g in range(GROUPS)
                ]
                for g in range(GROUPS):
                    vout[j, pl.ds(g * 16, 16)] = vals[g]
            return 0

        lax.fori_loop(0, ncols // 2, j_body, 0)

    for p in range(nbuf):
        in_copy(p, p).start()

    def t_body(t, _):
        for p in range(nbuf):
            i = t * nbuf + p

            @pl.when(i < n_blk)
            def _():
                in_copy(i, p).wait()

                @pl.when(i >= nbuf)
                def _():
                    out_copy(i - nbuf, p).wait()

                transpose(vins[p], vouts[p], 128)
                out_copy(i, p).start()

                @pl.when(i + nbuf < n_blk)
                def _():
                    in_copy(i + nbuf, p).start()

        return 0

    max_groups = (FULL_BLOCKS // NW + 1 + nbuf - 1) // nbuf
    lax.fori_loop(0, max_groups, t_body, 0)
    for p in range(nbuf):
        out_copy(p, p).wait()

    @pl.when(wid == NW - 1)
    def _():
        pltpu.sync_copy(tailt_hbm, vtail)
        transpose(vtail, vout0, REM)
        pltpu.sync_copy(
            vout0.at[pl.ds(0, REM)],
            out_hbm.at[pl.ds(FULL_BLOCKS * 128, REM)],
        )


@functools.partial(
    pl.kernel,
    out_type=jax.ShapeDtypeStruct((B, D), jnp.float32),
    mesh=_mesh,
    scratch_types=[
        pltpu.VMEM((L, B_PER_W), jnp.int32),
        pltpu.VMEM((B_PER_W, 2 * D), jnp.float32),
        pltpu.VMEM((B_PER_W, 2 * D), jnp.float32),
        pltpu.VMEM((B_PER_W, 2 * D), jnp.float32),
        pltpu.VMEM((B_PER_W, 2 * D), jnp.float32),
        pltpu.VMEM((B_PER_W, D), jnp.float32),
        pltpu.SemaphoreType.DMA,
        pltpu.SemaphoreType.DMA,
        pltpu.SemaphoreType.DMA,
        pltpu.SemaphoreType.DMA,
    ],
)
def _pool(xt_hbm, embp_hbm, out_hbm, idx_v, buf0, buf1, buf2, buf3, acc_v,
          sem0, sem1, sem2, sem3):
    wid = lax.axis_index("s") * NC + lax.axis_index("c")
    base = wid * B_PER_W

    pltpu.sync_copy(xt_hbm.at[:, pl.ds(base, B_PER_W)], idx_v)

    bufs = (buf0, buf1, buf2, buf3)
    sems = (sem0, sem1, sem2, sem3)
    pool_nbuf = 4

    def start(pos, half):
        pltpu.make_async_copy(
            embp_hbm.at[idx_v.at[pos]], bufs[half], sems[half]
        ).start()

    def wait(pos, half):
        pltpu.make_async_copy(
            embp_hbm.at[idx_v.at[pos]], bufs[half], sems[half]
        ).wait()

    zero = jnp.zeros((16,), jnp.float32)

    def zero_body(r, _):
        for u in range(ROW_UNROLL):
            row = r * ROW_UNROLL + u
            for g in range(GROUPS):
                acc_v[row, pl.ds(g * 16, 16)] = zero
        return 0

    lax.fori_loop(0, B_PER_W // ROW_UNROLL, zero_body, 0)

    for p in range(pool_nbuf):
        start(p, p)

    def accum(buf):
        def row_body(r, _):
            for u in range(ROW_UNROLL):
                row = r * ROW_UNROLL + u
                for g in range(GROUPS):
                    plsc.addupdate(
                        acc_v.at[row, pl.ds(g * 16, 16)],
                        buf[row, pl.ds(g * 16, 16)],
                    )
            return 0

        lax.fori_loop(0, B_PER_W // ROW_UNROLL, row_body, 0)

    def pos_body(i, _):
        for half in range(pool_nbuf):
            pos = i * pool_nbuf + half
            wait(pos, half)
            accum(bufs[half])

            @pl.when(pos + pool_nbuf < L)
            def _():
                start(pos + pool_nbuf, half)

        return 0

    lax.fori_loop(0, L // pool_nbuf, pos_body, 0)

    pltpu.sync_copy(acc_v, out_hbm.at[pl.ds(base, B_PER_W)])


def _mlp_body(m_ref, vw_ref, vb_ref, ww_ref, wb_ref, out_ref):
    m = m_ref[...]
    h = jnp.dot(m, vw_ref[...], preferred_element_type=jnp.float32)
    h = jnp.maximum(h + vb_ref[...], 0.0)
    logits = jnp.dot(h, ww_ref[...], preferred_element_type=jnp.float32)
    logits = logits + wb_ref[...]
    mx = jnp.max(logits, axis=1, keepdims=True)
    s = logits - mx
    lse = jnp.log(jnp.sum(jnp.exp(s), axis=1, keepdims=True))
    out_ref[...] = s - lse


def _mlp(m, vw_t, vb, ww_t, wb):
    return pl.pallas_call(
        _mlp_body,
        out_shape=jax.ShapeDtypeStruct((B, NUM_CLASSES), jnp.float32),
    )(m, vw_t, vb, ww_t, wb)


@jax.jit
def kernel(x, emb, V_w, V_b, W_w, W_b):
    xt = x.astype(jnp.int32).T
    tailt = emb[VOCAB_N - REM:].T
    embp = _relayout(emb.T, tailt)
    m_sum = _pool(xt, embp)
    vw_t = V_w.T * jnp.float32(1.0 / L)
    return _mlp(m_sum, vw_t, V_b.reshape(1, HID), W_w.T, W_b.reshape(1, NUM_CLASSES))


# speedup vs baseline: 1.1308x; 1.0012x over previous
import functools

import jax
import jax.numpy as jnp
from jax import lax
from jax.experimental import pallas as pl
from jax.experimental.pallas import tpu as pltpu
from jax.experimental.pallas import tpu_sc as plsc

B = 4096
L = 200
D = 64
HID = 128
NUM_CLASSES = 2

NC = 2
NS = 16
NW = NC * NS
B_PER_W = B // NW
GROUPS = D // 16
ROW_UNROLL = 2

_mesh = plsc.VectorSubcoreMesh(core_axis_name="c", subcore_axis_name="s")

VOCAB_N = 1000000
FULL_BLOCKS = VOCAB_N // 128
REM = VOCAB_N - FULL_BLOCKS * 128


@functools.partial(
    pl.kernel,
    out_type=jax.ShapeDtypeStruct((VOCAB_N, 2 * D), jnp.float32),
    mesh=_mesh,
    compiler_params=pltpu.CompilerParams(needs_layout_passes=False),
    scratch_types=[
        pltpu.VMEM((D, 128), jnp.float32),
        pltpu.VMEM((D, 128), jnp.float32),
        pltpu.VMEM((D, 128), jnp.float32),
        pltpu.VMEM((D, 128), jnp.float32),
        pltpu.VMEM((128, 2 * D), jnp.float32),
        pltpu.VMEM((128, 2 * D), jnp.float32),
        pltpu.VMEM((128, 2 * D), jnp.float32),
        pltpu.VMEM((128, 2 * D), jnp.float32),
        pltpu.VMEM((D, REM), jnp.float32),
        pltpu.VMEM((D, 128), jnp.float32),
        pltpu.SemaphoreType.DMA,
        pltpu.SemaphoreType.DMA,
        pltpu.SemaphoreType.DMA,
        pltpu.SemaphoreType.DMA,
        pltpu.SemaphoreType.DMA,
        pltpu.SemaphoreType.DMA,
        pltpu.SemaphoreType.DMA,
        pltpu.SemaphoreType.DMA,
    ],
)
def _relayout(embt_hbm, tailt_hbm, out_hbm, vin0, vin1, vin2, vin3,
              vout0, vout1, vout2, vout3, vtail, vskew,
              si0, si1, si2, si3, so0, so1, so2, so3):
    wid = lax.axis_index("s") * NC + lax.axis_index("c")
    n_blk = FULL_BLOCKS // NW + jnp.where(wid < FULL_BLOCKS % NW, 1, 0)
    first = wid * (FULL_BLOCKS // NW) + jnp.minimum(wid, FULL_BLOCKS % NW)

    vins = (vin0, vin1, vin2, vin3)
    vouts = (vout0, vout1, vout2, vout3)
    sis = (si0, si1, si2, si3)
    sos = (so0, so1, so2, so3)
    nbuf = 4
    iota = lax.iota(jnp.int32, 16)

    def in_copy(i, p):
        off = pl.multiple_of((first + i) * 128, 128)
        return pltpu.make_async_copy(
            embt_hbm.at[:, pl.ds(off, 128)], vins[p], sis[p]
        )

    def out_copy(i, p):
        off = pl.multiple_of((first + i) * 128, 128)
        return pltpu.make_async_copy(
            vouts[p], out_hbm.at[pl.ds(off, 128)], sos[p]
        )

    def transpose(vin, vout, ncols):
        rvs = [iota + g * 16 for g in range(GROUPS)]

        def skew_body(dt, _):
            for du in range(4):
                d = dt * 4 + du
                rot = (iota + d) & 15
                vals = [
                    vin[d, pl.ds(cg * 16, 16)].at[rot].get(
                        mode="promise_in_bounds")
                    for cg in range(ncols // 16)
                ]
                for cg in range(ncols // 16):
                    vskew[d, pl.ds(cg * 16, 16)] = vals[cg]
            return 0

        lax.fori_loop(0, D // 4, skew_body, 0)

        def j_body(jt, _):
            for ju in range(2):
                j = jt * 2 + ju
                cvec = ((j - iota) & 15) + (j & 0x70)
                vals = [
                    plsc.load_gather(vskew, [rvs[g], cvec])
                    for g in range(GROUPS)
                ]
                for g in range(GROUPS):
                    vout[j, pl.ds(g * 16, 16)] = vals[g]
            return 0

        lax.fori_loop(0, ncols // 2, j_body, 0)

    for p in range(nbuf):
        in_copy(p, p).start()

    def t_body(t, _):
        for p in range(nbuf):
            i = t * nbuf + p

            @pl.when(i < n_blk)
            def _():
                in_copy(i, p).wait()

                @pl.when(i >= nbuf)
                def _():
                    out_copy(i - nbuf, p).wait()

                transpose(vins[p], vouts[p], 128)
                out_copy(i, p).start()

                @pl.when(i + nbuf < n_blk)
                def _():
                    in_copy(i + nbuf, p).start()

        return 0

    max_groups = (FULL_BLOCKS // NW + 1 + nbuf - 1) // nbuf
    lax.fori_loop(0, max_groups, t_body, 0)
    for p in range(nbuf):
        out_copy(p, p).wait()

    @pl.when(wid == NW - 1)
    def _():
        pltpu.sync_copy(tailt_hbm, vtail)
        transpose(vtail, vout0, REM)
        pltpu.sync_copy(
            vout0.at[pl.ds(0, REM)],
            out_hbm.at[pl.ds(FULL_BLOCKS * 128, REM)],
        )


@functools.partial(
    pl.kernel,
    out_type=jax.ShapeDtypeStruct((B, D), jnp.float32),
    mesh=_mesh,
    scratch_types=[
        pltpu.VMEM((L, B_PER_W), jnp.int32),
        pltpu.VMEM((B_PER_W, 2 * D), jnp.float32),
        pltpu.VMEM((B_PER_W, 2 * D), jnp.float32),
        pltpu.VMEM((B_PER_W, 2 * D), jnp.float32),
        pltpu.VMEM((B_PER_W, 2 * D), jnp.float32),
        pltpu.VMEM((B_PER_W, D), jnp.float32),
        pltpu.SemaphoreType.DMA,
        pltpu.SemaphoreType.DMA,
        pltpu.SemaphoreType.DMA,
        pltpu.SemaphoreType.DMA,
    ],
)
def _pool(xt_hbm, embp_hbm, out_hbm, idx_v, buf0, buf1, buf2, buf3, acc_v,
          sem0, sem1, sem2, sem3):
    wid = lax.axis_index("s") * NC + lax.axis_index("c")
    base = wid * B_PER_W

    pltpu.sync_copy(xt_hbm.at[:, pl.ds(base, B_PER_W)], idx_v)

    bufs = (buf0, buf1, buf2, buf3)
    sems = (sem0, sem1, sem2, sem3)
    pool_nbuf = 4

    def start(pos, half):
        pltpu.make_async_copy(
            embp_hbm.at[idx_v.at[pos]], bufs[half], sems[half]
        ).start()

    def wait(pos, half):
        pltpu.make_async_copy(
            embp_hbm.at[idx_v.at[pos]], bufs[half], sems[half]
        ).wait()

    zero = jnp.zeros((16,), jnp.float32)

    def zero_body(r, _):
        for u in range(ROW_UNROLL):
            row = r * ROW_UNROLL + u
            for g in range(GROUPS):
                acc_v[row, pl.ds(g * 16, 16)] = zero
        return 0

    lax.fori_loop(0, B_PER_W // ROW_UNROLL, zero_body, 0)

    for p in range(pool_nbuf):
        start(p, p)

    def accum(buf):
        def row_body(r, _):
            for u in range(ROW_UNROLL):
                row = r * ROW_UNROLL + u
                for g in range(GROUPS):
                    plsc.addupdate(
                        acc_v.at[row, pl.ds(g * 16, 16)],
                        buf[row, pl.ds(g * 16, 16)],
                    )
            return 0

        lax.fori_loop(0, B_PER_W // ROW_UNROLL, row_body, 0)

    def pos_body(i, _):
        for half in range(pool_nbuf):
            pos = i * pool_nbuf + half
            wait(pos, half)
            accum(bufs[half])

            @pl.when(pos + pool_nbuf < L)
            def _():
                start(pos + pool_nbuf, half)

        return 0

    lax.fori_loop(0, L // pool_nbuf, pos_body, 0)

    pltpu.sync_copy(acc_v, out_hbm.at[pl.ds(base, B_PER_W)])


def _mlp_body(m_ref, vw_ref, vb_ref, ww_ref, wb_ref, out_ref):
    m = m_ref[...]
    h = jnp.dot(m, vw_ref[...], preferred_element_type=jnp.float32)
    h = jnp.maximum(h + vb_ref[...], 0.0)
    logits = jnp.dot(h, ww_ref[...], preferred_element_type=jnp.float32)
    logits = logits + wb_ref[...]
    mx = jnp.max(logits, axis=1, keepdims=True)
    s = logits - mx
    lse = jnp.log(jnp.sum(jnp.exp(s), axis=1, keepdims=True))
    out_ref[...] = s - lse


def _mlp(m, vw_t, vb, ww_t, wb):
    return pl.pallas_call(
        _mlp_body,
        out_shape=jax.ShapeDtypeStruct((B, NUM_CLASSES), jnp.float32),
    )(m, vw_t, vb, ww_t, wb)


@jax.jit
def kernel(x, emb, V_w, V_b, W_w, W_b):
    xt = x.astype(jnp.int32).T
    tailt = emb[VOCAB_N - REM:].T
    embp = _relayout(emb.T, tailt)
    m_sum = _pool(xt, embp)
    vw_t = V_w.T * jnp.float32(1.0 / L)
    return _mlp(m_sum, vw_t, V_b.reshape(1, HID), W_w.T, W_b.reshape(1, NUM_CLASSES))
